# Initial kernel scaffold; baseline (speedup 1.0000x reference)
import jax
import jax.numpy as jnp
from jax.experimental import pallas as pl


def kernel(x, emb_table, lin_table, lin_bias, gamma_fm, beta_fm, W1, b1, gamma1, beta1, W2, b2):
    raise NotImplementedError("write your pallas kernel here")


# speedup vs baseline: 1.0098x; 1.0098x over previous
import functools

import jax
import jax.numpy as jnp
import numpy as np
from jax import lax
from jax.experimental import pallas as pl
from jax.experimental.pallas import tpu as pltpu
from jax.experimental.pallas import tpu_sc as plsc

NF = 26
ED = 16
B = 16384
HID = 64
EPS = 1e-5
_OFFS = (np.arange(NF, dtype=np.int32) * 40000)

NC = 2
NS = 16
NW = NC * NS
C = 128
BPW = B // NW
NCH = BPW // C


def _sc_body(emb_hbm, lin_hbm, idx_hbm, cross_hbm, lsum_hbm,
             idx_v, emb_v, lin_v, cross_v, lsum_v, sem):
    wid = lax.axis_index("s") * NC + lax.axis_index("c")

    def chunk_body(c, carry):
        base = (wid * NCH + c) * C
        pltpu.sync_copy(idx_hbm.at[wid, c], idx_v)

        def gather_f(f, carry2):
            pltpu.async_copy(emb_hbm.at[idx_v.at[f]], emb_v.at[f], sem).wait()
            pltpu.async_copy(lin_hbm.at[idx_v.at[f]], lin_v.at[f], sem).wait()
            return carry2

        lax.fori_loop(0, NF, gather_f, 0)

        def sample_body(i, carry2):
            s = jnp.zeros((ED,), jnp.float32)
            sq = jnp.zeros((ED,), jnp.float32)
            for f in range(NF):
                v = emb_v[f, i, :]
                s = s + v
                sq = sq + v * v
            cross_v[i, :] = 0.5 * (s * s - sq)
            return carry2

        lax.fori_loop(0, C, sample_body, 0)

        def lin_body(j, carry2):
            acc = jnp.zeros((16,), jnp.float32)
            for f in range(NF):
                acc = acc + lin_v[f, pl.ds(j * 16, 16)]
            lsum_v[pl.ds(j * 16, 16)] = acc
            return carry2

        lax.fori_loop(0, C // 16, lin_body, 0)
        pltpu.sync_copy(cross_v, cross_hbm.at[pl.ds(base, C)])
        pltpu.sync_copy(lsum_v, lsum_hbm.at[pl.ds(base, C)])
        return carry

    lax.fori_loop(0, NCH, chunk_body, 0)


_sc_call = functools.partial(
    pl.kernel,
    out_type=(
        jax.ShapeDtypeStruct((B, ED), jnp.float32),
        jax.ShapeDtypeStruct((B,), jnp.float32),
    ),
    mesh=plsc.VectorSubcoreMesh(core_axis_name="c", subcore_axis_name="s"),
    compiler_params=pltpu.CompilerParams(use_tc_tiling_on_sc=False),
    scratch_types=[
        pltpu.VMEM((NF, C), jnp.int32),
        pltpu.VMEM((NF, C, ED), jnp.float32),
        pltpu.VMEM((NF, C), jnp.float32),
        pltpu.VMEM((C, ED), jnp.float32),
        pltpu.VMEM((C,), jnp.float32),
        pltpu.SemaphoreType.DMA,
    ],
)(_sc_body)


def _tc_body(cross_ref, lsum_ref, lin_bias_ref, gfm_ref, bfm_ref,
             w1_ref, b1_ref, g1_ref, be1_ref, w2_ref, b2_ref, out_ref):
    cross = cross_ref[:]
    mu = jnp.mean(cross, axis=0, keepdims=True)
    var = jnp.mean((cross - mu) ** 2, axis=0, keepdims=True)
    crossn = gfm_ref[:] * (cross - mu) / jnp.sqrt(var + EPS) + bfm_ref[:]
    h = jnp.dot(crossn, w1_ref[:], preferred_element_type=jnp.float32) + b1_ref[:]
    mu1 = jnp.mean(h, axis=0, keepdims=True)
    var1 = jnp.mean((h - mu1) ** 2, axis=0, keepdims=True)
    h = jax.nn.relu(g1_ref[:] * (h - mu1) / jnp.sqrt(var1 + EPS) + be1_ref[:])
    mlp = jnp.dot(h, w2_ref[:], preferred_element_type=jnp.float32)
    z = lsum_ref[:] + mlp + lin_bias_ref[0] + b2_ref[0]
    out_ref[:] = 1.0 / (1.0 + jnp.exp(-z))


def kernel(x, emb_table, lin_table, lin_bias, gamma_fm, beta_fm,
           W1, b1, gamma1, beta1, W2, b2):
    idx = x.astype(jnp.int32) + jnp.asarray(_OFFS)[None, :]
    idx4 = idx.reshape(NW, NCH, C, NF).transpose(0, 1, 3, 2)
    cross, lsum = _sc_call(emb_table, lin_table.reshape(-1), idx4)

    out = pl.pallas_call(
        _tc_body,
        out_shape=jax.ShapeDtypeStruct((B, 1), jnp.float32),
    )(cross, lsum.reshape(B, 1), lin_bias, gamma_fm, beta_fm,
      W1, b1, gamma1, beta1, W2, b2)
    return out[:, 0]


# speedup vs baseline: 1.1936x; 1.1820x over previous
import functools

import jax
import jax.numpy as jnp
import numpy as np
from jax import lax
from jax.experimental import pallas as pl
from jax.experimental.pallas import tpu as pltpu
from jax.experimental.pallas import tpu_sc as plsc

NF = 26
ED = 16
B = 16384
HID = 64
EPS = 1e-5
_OFFS = (np.arange(NF, dtype=np.int32) * 40000)

NC = 2
NS = 16
NW = NC * NS
C = 128
BPW = B // NW
NCH = BPW // C
FLAT = C * NF
NPC = FLAT // 128


def _sc_body(emb_hbm, lin_hbm, idx_hbm, cross_hbm, linrows_hbm,
             idx_v, emb_v, lin_v, cross_v, sem_e, sem_l):
    wid = lax.axis_index("s") * NC + lax.axis_index("c")

    def chunk_body(c, carry):
        base = (wid * NCH + c) * C
        pltpu.sync_copy(idx_hbm.at[wid, c], idx_v)

        def issue(j, cy):
            pltpu.async_copy(emb_hbm.at[idx_v.at[j]],
                             emb_v.at[pl.ds(j * 128, 128)], sem_e)
            pltpu.async_copy(lin_hbm.at[idx_v.at[j]],
                             lin_v.at[pl.ds(j * 128, 128)], sem_l)
            return cy

        lax.fori_loop(0, NPC, issue, 0)

        def drain(j, cy):
            pltpu.make_async_copy(emb_hbm.at[idx_v.at[j]],
                                  emb_v.at[pl.ds(j * 128, 128)], sem_e).wait()
            pltpu.make_async_copy(lin_hbm.at[idx_v.at[j]],
                                  lin_v.at[pl.ds(j * 128, 128)], sem_l).wait()
            return cy

        lax.fori_loop(0, NPC, drain, 0)

        def sample_body(i, cy):
            rb = i * NF
            s = jnp.zeros((ED,), jnp.float32)
            sq = jnp.zeros((ED,), jnp.float32)
            for f in range(NF):
                v = emb_v[rb + f, :]
                s = s + v
                sq = sq + v * v
            cross_v[i, :] = 0.5 * (s * s - sq)
            return cy

        lax.fori_loop(0, C, sample_body, 0)
        pltpu.sync_copy(cross_v, cross_hbm.at[pl.ds(base, C)])
        pltpu.sync_copy(lin_v, linrows_hbm.at[pl.ds(base * NF, FLAT)])
        return carry

    lax.fori_loop(0, NCH, chunk_body, 0)


_sc_call = functools.partial(
    pl.kernel,
    out_type=(
        jax.ShapeDtypeStruct((B, ED), jnp.float32),
        jax.ShapeDtypeStruct((B * NF,), jnp.float32),
    ),
    mesh=plsc.VectorSubcoreMesh(core_axis_name="c", subcore_axis_name="s"),
    compiler_params=pltpu.CompilerParams(use_tc_tiling_on_sc=False),
    scratch_types=[
        pltpu.VMEM((NPC, 128), jnp.int32),
        pltpu.VMEM((FLAT, ED), jnp.float32),
        pltpu.VMEM((FLAT,), jnp.float32),
        pltpu.VMEM((C, ED), jnp.float32),
        pltpu.SemaphoreType.DMA,
        pltpu.SemaphoreType.DMA,
    ],
)(_sc_body)


def _tc_body(cross_ref, linrows_ref, lin_bias_ref, gfm_ref, bfm_ref,
             w1_ref, b1_ref, g1_ref, be1_ref, w2_ref, b2_ref, out_ref):
    lsum = jnp.sum(linrows_ref[:], axis=1, keepdims=True)
    cross = cross_ref[:]
    mu = jnp.mean(cross, axis=0, keepdims=True)
    var = jnp.mean((cross - mu) ** 2, axis=0, keepdims=True)
    crossn = gfm_ref[:] * (cross - mu) / jnp.sqrt(var + EPS) + bfm_ref[:]
    h = jnp.dot(crossn, w1_ref[:], preferred_element_type=jnp.float32) + b1_ref[:]
    mu1 = jnp.mean(h, axis=0, keepdims=True)
    var1 = jnp.mean((h - mu1) ** 2, axis=0, keepdims=True)
    h = jax.nn.relu(g1_ref[:] * (h - mu1) / jnp.sqrt(var1 + EPS) + be1_ref[:])
    mlp = jnp.dot(h, w2_ref[:], preferred_element_type=jnp.float32)
    z = lsum + mlp + lin_bias_ref[0] + b2_ref[0]
    out_ref[:] = 1.0 / (1.0 + jnp.exp(-z))


def kernel(x, emb_table, lin_table, lin_bias, gamma_fm, beta_fm,
           W1, b1, gamma1, beta1, W2, b2):
    idx = x.astype(jnp.int32) + jnp.asarray(_OFFS)[None, :]
    idx4 = idx.reshape(NW, NCH, NPC, 128)
    cross, linrows = _sc_call(emb_table, lin_table.reshape(-1), idx4)

    out = pl.pallas_call(
        _tc_body,
        out_shape=jax.ShapeDtypeStruct((B, 1), jnp.float32),
    )(cross, linrows.reshape(B, NF), lin_bias, gamma_fm, beta_fm,
      W1, b1, gamma1, beta1, W2, b2)
    return out[:, 0]


# speedup vs baseline: 1.4686x; 1.2304x over previous
import functools

import jax
import jax.numpy as jnp
import numpy as np
from jax import lax
from jax.experimental import pallas as pl
from jax.experimental.pallas import tpu as pltpu
from jax.experimental.pallas import tpu_sc as plsc

NF = 26
ED = 16
B = 16384
HID = 64
EPS = 1e-5
_OFFS = (np.arange(NF, dtype=np.int32) * 40000)

NC = 2
NS = 16
NW = NC * NS
C = 128
BPW = B // NW
NCH = BPW // C
FLAT = C * NF
NPC = FLAT // 128

TOTAL = 40000 * NF
CB = 8320
Q = CB // 8
TBLK = TOTAL // CB


def _tp_body(in_ref, out_ref):
    y = jnp.transpose(in_ref[:])
    for m in range(8):
        out_ref[:, pl.ds(16 * m, 16)] = y[m * Q:(m + 1) * Q, :]


_tp_call = pl.pallas_call(
    _tp_body,
    grid=(TBLK,),
    in_specs=[pl.BlockSpec((16, CB), lambda g: (0, g))],
    out_specs=pl.BlockSpec((Q, 128), lambda g: (g, 0)),
    out_shape=jax.ShapeDtypeStruct((TOTAL // 8, 128), jnp.float32),
)


def _sc_body(emb_hbm, lin_hbm, idx_hbm, cross_hbm, linrows_hbm,
             pidx_v, idx_v, emb_v, lin_v, cross_v, sem_e, sem_l):
    wid = lax.axis_index("s") * NC + lax.axis_index("c")

    def chunk_body(c, carry):
        base = (wid * NCH + c) * C
        pltpu.sync_copy(idx_hbm.at[0, wid, c], pidx_v)
        pltpu.sync_copy(idx_hbm.at[1, wid, c], idx_v)

        def issue(j, cy):
            pltpu.async_copy(emb_hbm.at[pidx_v.at[j]],
                             emb_v.at[pl.ds(j * 128, 128)], sem_e)
            pltpu.async_copy(lin_hbm.at[idx_v.at[j]],
                             lin_v.at[pl.ds(j * 128, 128)], sem_l)
            return cy

        lax.fori_loop(0, NPC, issue, 0)

        def drain(j, cy):
            pltpu.make_async_copy(emb_hbm.at[pidx_v.at[j]],
                                  emb_v.at[pl.ds(j * 128, 128)], sem_e).wait()
            pltpu.make_async_copy(lin_hbm.at[idx_v.at[j]],
                                  lin_v.at[pl.ds(j * 128, 128)], sem_l).wait()
            return cy

        lax.fori_loop(0, NPC, drain, 0)

        def sample_body(i, cy):
            rb = i * NF
            s = jnp.zeros((ED,), jnp.float32)
            sq = jnp.zeros((ED,), jnp.float32)
            for f in range(NF):
                v = emb_v[rb + f, :]
                s = s + v
                sq = sq + v * v
            cross_v[i, :] = 0.5 * (s * s - sq)
            return cy

        lax.fori_loop(0, C, sample_body, 0)
        pltpu.sync_copy(cross_v, cross_hbm.at[pl.ds(base, C)])
        pltpu.sync_copy(lin_v, linrows_hbm.at[pl.ds(base * NF, FLAT)])
        return carry

    lax.fori_loop(0, NCH, chunk_body, 0)


_sc_call = functools.partial(
    pl.kernel,
    out_type=(
        jax.ShapeDtypeStruct((B, ED), jnp.float32),
        jax.ShapeDtypeStruct((B * NF,), jnp.float32),
    ),
    mesh=plsc.VectorSubcoreMesh(core_axis_name="c", subcore_axis_name="s"),
    compiler_params=pltpu.CompilerParams(use_tc_tiling_on_sc=False),
    scratch_types=[
        pltpu.VMEM((NPC, 128), jnp.int32),
        pltpu.VMEM((NPC, 128), jnp.int32),
        pltpu.VMEM((FLAT, ED), jnp.float32),
        pltpu.VMEM((FLAT,), jnp.float32),
        pltpu.VMEM((C, ED), jnp.float32),
        pltpu.SemaphoreType.DMA,
        pltpu.SemaphoreType.DMA,
    ],
)(_sc_body)


def _tc_body(cross_ref, linrows_ref, lin_bias_ref, gfm_ref, bfm_ref,
             w1_ref, b1_ref, g1_ref, be1_ref, w2_ref, b2_ref, out_ref):
    lsum = jnp.sum(linrows_ref[:], axis=1, keepdims=True)
    cross = cross_ref[:]
    mu = jnp.mean(cross, axis=0, keepdims=True)
    var = jnp.mean((cross - mu) ** 2, axis=0, keepdims=True)
    crossn = gfm_ref[:] * (cross - mu) / jnp.sqrt(var + EPS) + bfm_ref[:]
    h = jnp.dot(crossn, w1_ref[:], preferred_element_type=jnp.float32) + b1_ref[:]
    mu1 = jnp.mean(h, axis=0, keepdims=True)
    var1 = jnp.mean((h - mu1) ** 2, axis=0, keepdims=True)
    h = jax.nn.relu(g1_ref[:] * (h - mu1) / jnp.sqrt(var1 + EPS) + be1_ref[:])
    mlp = jnp.dot(h, w2_ref[:], preferred_element_type=jnp.float32)
    z = lsum + mlp + lin_bias_ref[0] + b2_ref[0]
    out_ref[:] = 1.0 / (1.0 + jnp.exp(-z))


def kernel(x, emb_table, lin_table, lin_bias, gamma_fm, beta_fm,
           W1, b1, gamma1, beta1, W2, b2):
    emb_flat = _tp_call(emb_table.T).reshape(TOTAL, ED)

    idx = x.astype(jnp.int32) + jnp.asarray(_OFFS)[None, :]
    g = idx // CB
    rem = idx - g * CB
    pidx = g * CB + (rem % Q) * 8 + rem // Q
    idx5 = jnp.stack([pidx, idx]).reshape(2, NW, NCH, NPC, 128)
    cross, linrows = _sc_call(emb_flat, lin_table.reshape(-1), idx5)

    out = pl.pallas_call(
        _tc_body,
        out_shape=jax.ShapeDtypeStruct((B, 1), jnp.float32),
    )(cross, linrows.reshape(B, NF), lin_bias, gamma_fm, beta_fm,
      W1, b1, gamma1, beta1, W2, b2)
    return out[:, 0]


# speedup vs baseline: 2.2522x; 1.5335x over previous
import functools

import jax
import jax.numpy as jnp
import numpy as np
from jax import lax
from jax.experimental import pallas as pl
from jax.experimental.pallas import tpu as pltpu
from jax.experimental.pallas import tpu_sc as plsc

NF = 26
ED = 16
B = 16384
HID = 64
EPS = 1e-5
_OFFS = (np.arange(NF, dtype=np.int32) * 40000)

NC = 2
NS = 16
NW = NC * NS
C = 128
BPW = B // NW
NCH = BPW // C
FLAT = C * NF
NPC = FLAT // 128

TOTAL = 40000 * NF
CB = 8320
Q = CB // 8
TBLK = TOTAL // CB


def _tp_body(in_ref, out_ref):
    for s in range(8):
        xs = jnp.concatenate(
            [in_ref[:, (s * 8 + a) * 128:(s * 8 + a + 1) * 128]
             for a in range(8)], axis=0)
        out_ref[pl.ds(128 * s, 128), :] = jnp.transpose(xs)
    yl = jnp.transpose(in_ref[:, 64 * 128:65 * 128])
    for m in range(8):
        out_ref[pl.ds(1024, 16), pl.ds(16 * m, 16)] = yl[16 * m:16 * (m + 1), :]


_tp_call = pl.pallas_call(
    _tp_body,
    grid=(TBLK,),
    in_specs=[pl.BlockSpec((16, CB), lambda g: (0, g))],
    out_specs=pl.BlockSpec((Q, 128), lambda g: (g, 0)),
    out_shape=jax.ShapeDtypeStruct((TOTAL // 8, 128), jnp.float32),
)


def _sc_body(emb_hbm, lin_hbm, idx_hbm, cross_hbm, linrows_hbm,
             pidx_v, idx_v, emb_v, lin_v, cross_v, sem_e, sem_l):
    wid = lax.axis_index("s") * NC + lax.axis_index("c")

    def chunk_body(c, carry):
        base = (wid * NCH + c) * C
        pltpu.sync_copy(idx_hbm.at[0, wid, c], pidx_v)
        pltpu.sync_copy(idx_hbm.at[1, wid, c], idx_v)

        def issue(j, cy):
            pltpu.async_copy(emb_hbm.at[pidx_v.at[j]],
                             emb_v.at[pl.ds(j * 128, 128)], sem_e)
            pltpu.async_copy(lin_hbm.at[idx_v.at[j]],
                             lin_v.at[pl.ds(j * 128, 128)], sem_l)
            return cy

        lax.fori_loop(0, NPC, issue, 0)

        def drain(j, cy):
            pltpu.make_async_copy(emb_hbm.at[pidx_v.at[j]],
                                  emb_v.at[pl.ds(j * 128, 128)], sem_e).wait()
            pltpu.make_async_copy(lin_hbm.at[idx_v.at[j]],
                                  lin_v.at[pl.ds(j * 128, 128)], sem_l).wait()
            return cy

        lax.fori_loop(0, NPC, drain, 0)

        def sample_body(i, cy):
            rb = i * NF
            s = jnp.zeros((ED,), jnp.float32)
            sq = jnp.zeros((ED,), jnp.float32)
            for f in range(NF):
                v = emb_v[rb + f, :]
                s = s + v
                sq = sq + v * v
            cross_v[i, :] = 0.5 * (s * s - sq)
            return cy

        lax.fori_loop(0, C, sample_body, 0)
        pltpu.sync_copy(cross_v, cross_hbm.at[pl.ds(base, C)])
        pltpu.sync_copy(lin_v, linrows_hbm.at[pl.ds(base * NF, FLAT)])
        return carry

    lax.fori_loop(0, NCH, chunk_body, 0)


_sc_call = functools.partial(
    pl.kernel,
    out_type=(
        jax.ShapeDtypeStruct((B, ED), jnp.float32),
        jax.ShapeDtypeStruct((B * NF,), jnp.float32),
    ),
    mesh=plsc.VectorSubcoreMesh(core_axis_name="c", subcore_axis_name="s"),
    compiler_params=pltpu.CompilerParams(use_tc_tiling_on_sc=False),
    scratch_types=[
        pltpu.VMEM((NPC, 128), jnp.int32),
        pltpu.VMEM((NPC, 128), jnp.int32),
        pltpu.VMEM((FLAT, ED), jnp.float32),
        pltpu.VMEM((FLAT,), jnp.float32),
        pltpu.VMEM((C, ED), jnp.float32),
        pltpu.SemaphoreType.DMA,
        pltpu.SemaphoreType.DMA,
    ],
)(_sc_body)


def _tc_body(cross_ref, linrows_ref, lin_bias_ref, gfm_ref, bfm_ref,
             w1_ref, b1_ref, g1_ref, be1_ref, w2_ref, b2_ref, out_ref):
    lsum = jnp.sum(linrows_ref[:], axis=1, keepdims=True)
    cross = cross_ref[:]
    mu = jnp.mean(cross, axis=0, keepdims=True)
    var = jnp.mean((cross - mu) ** 2, axis=0, keepdims=True)
    crossn = gfm_ref[:] * (cross - mu) / jnp.sqrt(var + EPS) + bfm_ref[:]
    h = jnp.dot(crossn, w1_ref[:], preferred_element_type=jnp.float32) + b1_ref[:]
    mu1 = jnp.mean(h, axis=0, keepdims=True)
    var1 = jnp.mean((h - mu1) ** 2, axis=0, keepdims=True)
    h = jax.nn.relu(g1_ref[:] * (h - mu1) / jnp.sqrt(var1 + EPS) + be1_ref[:])
    mlp = jnp.dot(h, w2_ref[:], preferred_element_type=jnp.float32)
    z = lsum + mlp + lin_bias_ref[0] + b2_ref[0]
    out_ref[:] = 1.0 / (1.0 + jnp.exp(-z))


def kernel(x, emb_table, lin_table, lin_bias, gamma_fm, beta_fm,
           W1, b1, gamma1, beta1, W2, b2):
    emb_flat = _tp_call(emb_table.T).reshape(TOTAL, ED)

    idx = x.astype(jnp.int32) + jnp.asarray(_OFFS)[None, :]
    g = idx // CB
    rem = idx - g * CB
    tau = rem // 128
    lane = rem - tau * 128
    pidx_main = g * CB + (tau // 8) * 1024 + lane * 8 + tau % 8
    pidx_left = g * CB + 8192 + (lane % 16) * 8 + lane // 16
    pidx = jnp.where(tau < 64, pidx_main, pidx_left)
    idx5 = jnp.stack([pidx, idx]).reshape(2, NW, NCH, NPC, 128)
    cross, linrows = _sc_call(emb_flat, lin_table.reshape(-1), idx5)

    out = pl.pallas_call(
        _tc_body,
        out_shape=jax.ShapeDtypeStruct((B, 1), jnp.float32),
    )(cross, linrows.reshape(B, NF), lin_bias, gamma_fm, beta_fm,
      W1, b1, gamma1, beta1, W2, b2)
    return out[:, 0]


# speedup vs baseline: 3.5525x; 1.5774x over previous
import functools

import jax
import jax.numpy as jnp
from jax import lax
from jax.experimental import pallas as pl
from jax.experimental.pallas import tpu as pltpu
from jax.experimental.pallas import tpu_sc as plsc

NF = 26
ED = 16
B = 16384
HID = 64
EPS = 1e-5

NC = 2
NS = 16
NW = NC * NS
C = 128
BPW = B // NW
NCH = BPW // C
FLAT = C * NF

TOTAL = 40000 * NF
CB = 16384
TBLK = 64
PROWS = TBLK * 2048 * 8


def _tp_body(in_ref, out_ref):
    for s in range(16):
        xs = jnp.concatenate(
            [in_ref[:, (s * 8 + a) * 128:(s * 8 + a + 1) * 128]
             for a in range(8)], axis=0)
        out_ref[pl.ds(128 * s, 128), :] = jnp.transpose(xs)


_tp_call = pl.pallas_call(
    _tp_body,
    grid=(TBLK,),
    in_specs=[pl.BlockSpec((16, CB), lambda g: (0, g))],
    out_specs=pl.BlockSpec((2048, 128), lambda g: (g, 0)),
    out_shape=jax.ShapeDtypeStruct((TBLK * 2048, 128), jnp.float32),
)


def _sc1_body(xt_hbm, lin_hbm, lsum_hbm, pidx_hbm,
              x_v, ridx_v, pidx_v, lin_v, lsum_v, sem_l):
    wid = lax.axis_index("s") * NC + lax.axis_index("c")

    def chunk_body(c, carry):
        base = wid * BPW + c * C
        pltpu.sync_copy(xt_hbm.at[:, pl.ds(base, C)], x_v)
        for f in range(NF):
            off = f * 40000
            for k in range(C // 16):
                r = x_v[f, pl.ds(k * 16, 16)] + off
                p = (((r >> 10) << 10) + ((r & 127) << 3) + ((r >> 7) & 7))
                ridx_v[f, pl.ds(k * 16, 16)] = r
                pidx_v[f, pl.ds(k * 16, 16)] = p

        def issue(f, cy):
            pltpu.async_copy(lin_hbm.at[ridx_v.at[f]], lin_v.at[f], sem_l)
            return cy

        lax.fori_loop(0, NF, issue, 0)

        def drain(f, cy):
            pltpu.make_async_copy(lin_hbm.at[ridx_v.at[f]],
                                  lin_v.at[f], sem_l).wait()
            return cy

        lax.fori_loop(0, NF, drain, 0)

        for k in range(C // 16):
            acc = jnp.zeros((16,), jnp.float32)
            for f in range(NF):
                acc = acc + lin_v[f, pl.ds(k * 16, 16)]
            lsum_v[pl.ds(k * 16, 16)] = acc

        pltpu.sync_copy(lsum_v, lsum_hbm.at[pl.ds(base, C)])
        pltpu.sync_copy(pidx_v, pidx_hbm.at[wid, c])
        return carry

    lax.fori_loop(0, NCH, chunk_body, 0)


_sc1_call = functools.partial(
    pl.kernel,
    out_type=(
        jax.ShapeDtypeStruct((B,), jnp.float32),
        jax.ShapeDtypeStruct((NW, NCH, NF, C), jnp.int32),
    ),
    mesh=plsc.VectorSubcoreMesh(core_axis_name="c", subcore_axis_name="s"),
    compiler_params=pltpu.CompilerParams(use_tc_tiling_on_sc=False),
    scratch_types=[
        pltpu.VMEM((NF, C), jnp.int32),
        pltpu.VMEM((NF, C), jnp.int32),
        pltpu.VMEM((NF, C), jnp.int32),
        pltpu.VMEM((NF, C), jnp.float32),
        pltpu.VMEM((C,), jnp.float32),
        pltpu.SemaphoreType.DMA,
    ],
)(_sc1_body)


def _sc2_body(emb_hbm, pidx_hbm, cross_hbm, pidx_v, emb_v, cross_v, sem_e):
    wid = lax.axis_index("s") * NC + lax.axis_index("c")

    def chunk_body(c, carry):
        base = wid * BPW + c * C
        pltpu.sync_copy(pidx_hbm.at[wid, c], pidx_v)

        def issue(f, cy):
            pltpu.async_copy(emb_hbm.at[pidx_v.at[f]],
                             emb_v.at[pl.ds(f * C, C)], sem_e)
            return cy

        lax.fori_loop(0, NF, issue, 0)

        def drain(f, cy):
            pltpu.make_async_copy(emb_hbm.at[pidx_v.at[f]],
                                  emb_v.at[pl.ds(f * C, C)], sem_e).wait()
            return cy

        lax.fori_loop(0, NF, drain, 0)

        def sample_body(i, cy):
            s = jnp.zeros((ED,), jnp.float32)
            sq = jnp.zeros((ED,), jnp.float32)
            for f in range(NF):
                v = emb_v[f * C + i, :]
                s = s + v
                sq = sq + v * v
            cross_v[i, :] = 0.5 * (s * s - sq)
            return cy

        lax.fori_loop(0, C, sample_body, 0)
        pltpu.sync_copy(cross_v, cross_hbm.at[pl.ds(base, C)])
        return carry

    lax.fori_loop(0, NCH, chunk_body, 0)


_sc2_call = functools.partial(
    pl.kernel,
    out_type=jax.ShapeDtypeStruct((B, ED), jnp.float32),
    mesh=plsc.VectorSubcoreMesh(core_axis_name="c", subcore_axis_name="s"),
    compiler_params=pltpu.CompilerParams(use_tc_tiling_on_sc=False),
    scratch_types=[
        pltpu.VMEM((NF, C), jnp.int32),
        pltpu.VMEM((FLAT, ED), jnp.float32),
        pltpu.VMEM((C, ED), jnp.float32),
        pltpu.SemaphoreType.DMA,
    ],
)(_sc2_body)


def _tc_body(cross_ref, lsum_ref, lin_bias_ref, gfm_ref, bfm_ref,
             w1_ref, b1_ref, g1_ref, be1_ref, w2_ref, b2_ref, out_ref):
    cross = cross_ref[:]
    mu = jnp.mean(cross, axis=0, keepdims=True)
    var = jnp.mean((cross - mu) ** 2, axis=0, keepdims=True)
    crossn = gfm_ref[:] * (cross - mu) / jnp.sqrt(var + EPS) + bfm_ref[:]
    h = jnp.dot(crossn, w1_ref[:], preferred_element_type=jnp.float32) + b1_ref[:]
    mu1 = jnp.mean(h, axis=0, keepdims=True)
    var1 = jnp.mean((h - mu1) ** 2, axis=0, keepdims=True)
    h = jax.nn.relu(g1_ref[:] * (h - mu1) / jnp.sqrt(var1 + EPS) + be1_ref[:])
    mlp = jnp.sum(h * w2_ref[:], axis=1)
    z = lsum_ref[:] + mlp + lin_bias_ref[0] + b2_ref[0]
    out_ref[:] = 1.0 / (1.0 + jnp.exp(-z))


def kernel(x, emb_table, lin_table, lin_bias, gamma_fm, beta_fm,
           W1, b1, gamma1, beta1, W2, b2):
    emb_flat = _tp_call(emb_table.T).reshape(PROWS, ED)
    lsum, pidx = _sc1_call(x.astype(jnp.int32).T, lin_table.reshape(-1))
    cross = _sc2_call(emb_flat, pidx)

    out = pl.pallas_call(
        _tc_body,
        out_shape=jax.ShapeDtypeStruct((B,), jnp.float32),
    )(cross, lsum, lin_bias, gamma_fm, beta_fm,
      W1, b1, gamma1, beta1, W2.reshape(1, HID), b2)
    return out


# speedup vs baseline: 4.0600x; 1.1428x over previous
import functools

import jax
import jax.numpy as jnp
from jax import lax
from jax.experimental import pallas as pl
from jax.experimental.pallas import tpu as pltpu
from jax.experimental.pallas import tpu_sc as plsc

NF = 26
ED = 16
B = 16384
HID = 64
EPS = 1e-5

NC = 2
NS = 16
NW = NC * NS
C = 128
BPW = B // NW
NCH = BPW // C
FLAT = C * NF

TOTAL = 40000 * NF
CB = 32768
TBLK = 32
GRP = CB // 1024
PROWS = TBLK * CB


def _tp_body(in_ref, out_ref):
    for s in range(GRP):
        xs = jnp.concatenate(
            [in_ref[:, (s * 8 + a) * 128:(s * 8 + a + 1) * 128]
             for a in range(8)], axis=0)
        out_ref[pl.ds(128 * s, 128), :] = jnp.transpose(xs)


_tp_call = pl.pallas_call(
    _tp_body,
    grid=(TBLK,),
    in_specs=[pl.BlockSpec((16, CB), lambda g: (0, g))],
    out_specs=pl.BlockSpec((CB // 8, 128), lambda g: (g, 0)),
    out_shape=jax.ShapeDtypeStruct((TBLK * CB // 8, 128), jnp.float32),
)


def _sc1_body(xt_hbm, lin_hbm, lsum_hbm, pidx_hbm,
              x_v, ridx_v, pidx_v, lin_v, lsum_v, sem_l):
    wid = lax.axis_index("s") * NC + lax.axis_index("c")

    def chunk_body(c, carry):
        base = wid * BPW + c * C
        pltpu.sync_copy(xt_hbm.at[:, pl.ds(base, C)], x_v)
        for f in range(NF):
            off = f * 40000
            for k in range(C // 16):
                r = x_v[f, pl.ds(k * 16, 16)] + off
                p = (((r >> 10) << 10) + ((r & 127) << 3) + ((r >> 7) & 7))
                ridx_v[f, pl.ds(k * 16, 16)] = r
                pidx_v[f, pl.ds(k * 16, 16)] = p

        def issue(f, cy):
            pltpu.async_copy(lin_hbm.at[ridx_v.at[f]], lin_v.at[f], sem_l)
            return cy

        lax.fori_loop(0, NF, issue, 0)

        def drain(f, cy):
            pltpu.make_async_copy(lin_hbm.at[ridx_v.at[f]],
                                  lin_v.at[f], sem_l).wait()
            return cy

        lax.fori_loop(0, NF, drain, 0)

        for k in range(C // 16):
            acc = jnp.zeros((16,), jnp.float32)
            for f in range(NF):
                acc = acc + lin_v[f, pl.ds(k * 16, 16)]
            lsum_v[pl.ds(k * 16, 16)] = acc

        pltpu.sync_copy(lsum_v, lsum_hbm.at[pl.ds(base, C)])
        pltpu.sync_copy(pidx_v, pidx_hbm.at[wid, c])
        return carry

    lax.fori_loop(0, NCH, chunk_body, 0)


_sc1_call = functools.partial(
    pl.kernel,
    out_type=(
        jax.ShapeDtypeStruct((B,), jnp.float32),
        jax.ShapeDtypeStruct((NW, NCH, NF, C), jnp.int32),
    ),
    mesh=plsc.VectorSubcoreMesh(core_axis_name="c", subcore_axis_name="s"),
    compiler_params=pltpu.CompilerParams(use_tc_tiling_on_sc=False),
    scratch_types=[
        pltpu.VMEM((NF, C), jnp.int32),
        pltpu.VMEM((NF, C), jnp.int32),
        pltpu.VMEM((NF, C), jnp.int32),
        pltpu.VMEM((NF, C), jnp.float32),
        pltpu.VMEM((C,), jnp.float32),
        pltpu.SemaphoreType.DMA,
    ],
)(_sc1_body)


def _sc2_body(emb_hbm, pidx_hbm, cross_hbm,
              pidx_a, pidx_b, emb_a, emb_b, cross_v, sem_a, sem_b):
    wid = lax.axis_index("s") * NC + lax.axis_index("c")
    bufs = [(pidx_a, emb_a, sem_a), (pidx_b, emb_b, sem_b)]

    def stage(c, pv, ev, sem):
        pltpu.sync_copy(pidx_hbm.at[wid, c], pv)

        def issue(f, cy):
            pltpu.async_copy(emb_hbm.at[pv.at[f]],
                             ev.at[pl.ds(f * C, C)], sem)
            return cy

        lax.fori_loop(0, NF, issue, 0)

    def drain(pv, ev, sem):
        def dr(f, cy):
            pltpu.make_async_copy(emb_hbm.at[pv.at[f]],
                                  ev.at[pl.ds(f * C, C)], sem).wait()
            return cy

        lax.fori_loop(0, NF, dr, 0)

    stage(0, *bufs[0])
    for c in range(NCH):
        pv, ev, sem = bufs[c % 2]
        if c + 1 < NCH:
            stage(c + 1, *bufs[(c + 1) % 2])
        drain(pv, ev, sem)

        def sample_body(i, cy, ev=ev):
            s = jnp.zeros((ED,), jnp.float32)
            sq = jnp.zeros((ED,), jnp.float32)
            for f in range(NF):
                v = ev[f * C + i, :]
                s = s + v
                sq = sq + v * v
            cross_v[i, :] = 0.5 * (s * s - sq)
            return cy

        lax.fori_loop(0, C, sample_body, 0)
        pltpu.sync_copy(cross_v, cross_hbm.at[pl.ds(wid * BPW + c * C, C)])


_sc2_call = functools.partial(
    pl.kernel,
    out_type=jax.ShapeDtypeStruct((B, ED), jnp.float32),
    mesh=plsc.VectorSubcoreMesh(core_axis_name="c", subcore_axis_name="s"),
    compiler_params=pltpu.CompilerParams(use_tc_tiling_on_sc=False),
    scratch_types=[
        pltpu.VMEM((NF, C), jnp.int32),
        pltpu.VMEM((NF, C), jnp.int32),
        pltpu.VMEM((FLAT, ED), jnp.float32),
        pltpu.VMEM((FLAT, ED), jnp.float32),
        pltpu.VMEM((C, ED), jnp.float32),
        pltpu.SemaphoreType.DMA,
        pltpu.SemaphoreType.DMA,
    ],
)(_sc2_body)


def _tc_body(cross_ref, lsum_ref, lin_bias_ref, gfm_ref, bfm_ref,
             w1_ref, b1_ref, g1_ref, be1_ref, w2_ref, b2_ref, out_ref):
    cross = cross_ref[:]
    mu = jnp.mean(cross, axis=0, keepdims=True)
    var = jnp.mean((cross - mu) ** 2, axis=0, keepdims=True)
    crossn = gfm_ref[:] * (cross - mu) / jnp.sqrt(var + EPS) + bfm_ref[:]
    h = jnp.dot(crossn, w1_ref[:], preferred_element_type=jnp.float32) + b1_ref[:]
    mu1 = jnp.mean(h, axis=0, keepdims=True)
    var1 = jnp.mean((h - mu1) ** 2, axis=0, keepdims=True)
    h = jax.nn.relu(g1_ref[:] * (h - mu1) / jnp.sqrt(var1 + EPS) + be1_ref[:])
    mlp = jnp.sum(h * w2_ref[:], axis=1)
    z = lsum_ref[:] + mlp + lin_bias_ref[0] + b2_ref[0]
    out_ref[:] = 1.0 / (1.0 + jnp.exp(-z))


def kernel(x, emb_table, lin_table, lin_bias, gamma_fm, beta_fm,
           W1, b1, gamma1, beta1, W2, b2):
    emb_flat = _tp_call(emb_table.T).reshape(PROWS, ED)
    lsum, pidx = _sc1_call(x.astype(jnp.int32).T, lin_table.reshape(-1))
    cross = _sc2_call(emb_flat, pidx)

    out = pl.pallas_call(
        _tc_body,
        out_shape=jax.ShapeDtypeStruct((B,), jnp.float32),
    )(cross, lsum, lin_bias, gamma_fm, beta_fm,
      W1, b1, gamma1, beta1, W2.reshape(1, HID), b2)
    return out


# speedup vs baseline: 4.1703x; 1.0272x over previous
import functools

import jax
import jax.numpy as jnp
from jax import lax
from jax.experimental import pallas as pl
from jax.experimental.pallas import tpu as pltpu
from jax.experimental.pallas import tpu_sc as plsc

NF = 26
ED = 16
B = 16384
HID = 64
EPS = 1e-5

NC = 2
NS = 16
NW = NC * NS
C = 128
BPW = B // NW
NCH = BPW // C
FLAT = C * NF

TOTAL = 40000 * NF
CB = 32768
TBLK = 32
GRP = CB // 1024
PROWS = TBLK * CB


def _tp_body(in_ref, out_ref):
    for s in range(GRP):
        xs = jnp.concatenate(
            [in_ref[:, (s * 8 + a) * 128:(s * 8 + a + 1) * 128]
             for a in range(8)], axis=0)
        out_ref[pl.ds(128 * s, 128), :] = jnp.transpose(xs)


_tp_call = pl.pallas_call(
    _tp_body,
    grid=(TBLK,),
    in_specs=[pl.BlockSpec((16, CB), lambda g: (0, g))],
    out_specs=pl.BlockSpec((CB // 8, 128), lambda g: (g, 0)),
    out_shape=jax.ShapeDtypeStruct((TBLK * CB // 8, 128), jnp.float32),
)


def _sc1_body(xt_hbm, lin_hbm, lsum_hbm, pidx_hbm,
              x_v, ridx_v, pidx_v, lin_v, lsum_v, sem_l):
    wid = lax.axis_index("s") * NC + lax.axis_index("c")

    def chunk_body(c, carry):
        base = wid * BPW + c * C
        pltpu.sync_copy(xt_hbm.at[:, pl.ds(base, C)], x_v)
        for f in range(NF):
            off = f * 40000
            for k in range(C // 16):
                r = x_v[f, pl.ds(k * 16, 16)] + off
                p = (((r >> 10) << 10) + ((r & 127) << 3) + ((r >> 7) & 7))
                ridx_v[f, pl.ds(k * 16, 16)] = r
                pidx_v[f, pl.ds(k * 16, 16)] = p

        def issue(f, cy):
            pltpu.async_copy(lin_hbm.at[ridx_v.at[f]], lin_v.at[f], sem_l)
            return cy

        lax.fori_loop(0, NF, issue, 0)

        def drain(f, cy):
            pltpu.make_async_copy(lin_hbm.at[ridx_v.at[f]],
                                  lin_v.at[f], sem_l).wait()
            return cy

        lax.fori_loop(0, NF, drain, 0)

        for k in range(C // 16):
            acc = jnp.zeros((16,), jnp.float32)
            for f in range(NF):
                acc = acc + lin_v[f, pl.ds(k * 16, 16)]
            lsum_v[pl.ds(k * 16, 16)] = acc

        pltpu.sync_copy(lsum_v, lsum_hbm.at[pl.ds(base, C)])
        pltpu.sync_copy(pidx_v, pidx_hbm.at[wid, c])
        return carry

    lax.fori_loop(0, NCH, chunk_body, 0)


_sc1_call = functools.partial(
    pl.kernel,
    out_type=(
        jax.ShapeDtypeStruct((B,), jnp.float32),
        jax.ShapeDtypeStruct((NW, NCH, NF, C), jnp.int32),
    ),
    mesh=plsc.VectorSubcoreMesh(core_axis_name="c", subcore_axis_name="s"),
    compiler_params=pltpu.CompilerParams(use_tc_tiling_on_sc=False),
    scratch_types=[
        pltpu.VMEM((NF, C), jnp.int32),
        pltpu.VMEM((NF, C), jnp.int32),
        pltpu.VMEM((NF, C), jnp.int32),
        pltpu.VMEM((NF, C), jnp.float32),
        pltpu.VMEM((C,), jnp.float32),
        pltpu.SemaphoreType.DMA,
    ],
)(_sc1_body)


def _sc2_body(emb_hbm, pidx_hbm, cross_hbm,
              pidx_a, pidx_b, emb_a, emb_b, cross_v, sem_a, sem_b):
    wid = lax.axis_index("s") * NC + lax.axis_index("c")
    bufs = [(pidx_a, emb_a, sem_a), (pidx_b, emb_b, sem_b)]

    def stage(c, pv, ev, sem):
        pltpu.sync_copy(pidx_hbm.at[wid, c], pv)

        def issue(f, cy):
            pltpu.async_copy(emb_hbm.at[pv.at[f]],
                             ev.at[pl.ds(f * C, C)], sem)
            return cy

        lax.fori_loop(0, NF, issue, 0)

    def drain(pv, ev, sem):
        def dr(f, cy):
            pltpu.make_async_copy(emb_hbm.at[pv.at[f]],
                                  ev.at[pl.ds(f * C, C)], sem).wait()
            return cy

        lax.fori_loop(0, NF, dr, 0)

    stage(0, *bufs[0])
    for c in range(NCH):
        pv, ev, sem = bufs[c % 2]
        if c + 1 < NCH:
            stage(c + 1, *bufs[(c + 1) % 2])
        drain(pv, ev, sem)

        def sample_body(i, cy, ev=ev):
            s = jnp.zeros((ED,), jnp.float32)
            sq = jnp.zeros((ED,), jnp.float32)
            for f in range(NF):
                v = ev[f * C + i, :]
                s = s + v
                sq = sq + v * v
            cross_v[i, :] = 0.5 * (s * s - sq)
            return cy

        lax.fori_loop(0, C, sample_body, 0)
        pltpu.sync_copy(cross_v, cross_hbm.at[pl.ds(wid * BPW + c * C, C)])


_sc2_call = functools.partial(
    pl.kernel,
    out_type=jax.ShapeDtypeStruct((B, ED), jnp.float32),
    mesh=plsc.VectorSubcoreMesh(core_axis_name="c", subcore_axis_name="s"),
    compiler_params=pltpu.CompilerParams(use_tc_tiling_on_sc=False),
    scratch_types=[
        pltpu.VMEM((NF, C), jnp.int32),
        pltpu.VMEM((NF, C), jnp.int32),
        pltpu.VMEM((FLAT, ED), jnp.float32),
        pltpu.VMEM((FLAT, ED), jnp.float32),
        pltpu.VMEM((C, ED), jnp.float32),
        pltpu.SemaphoreType.DMA,
        pltpu.SemaphoreType.DMA,
    ],
)(_sc2_body)


def _bn_stats(v):
    ones_row = jnp.ones((1, B), jnp.float32)
    both = jnp.concatenate([v, v * v], axis=1)
    s = jnp.dot(ones_row, both, preferred_element_type=jnp.float32) / B
    n = v.shape[1]
    mu = s[:, :n]
    var = s[:, n:] - mu * mu
    return mu, var


def _tc_body(cross_ref, lsum_ref, lin_bias_ref, gfm_ref, bfm_ref,
             w1_ref, b1_ref, g1_ref, be1_ref, w2_ref, b2_ref, out_ref):
    cross = cross_ref[:]
    mu, var = _bn_stats(cross)
    crossn = gfm_ref[:] * (cross - mu) / jnp.sqrt(var + EPS) + bfm_ref[:]
    h = jnp.dot(crossn, w1_ref[:], preferred_element_type=jnp.float32) + b1_ref[:]
    mu1, var1 = _bn_stats(h)
    h = jax.nn.relu(g1_ref[:] * (h - mu1) / jnp.sqrt(var1 + EPS) + be1_ref[:])
    mlp = jnp.sum(h * w2_ref[:], axis=1)
    z = lsum_ref[:] + mlp + lin_bias_ref[0] + b2_ref[0]
    out_ref[:] = 1.0 / (1.0 + jnp.exp(-z))


def kernel(x, emb_table, lin_table, lin_bias, gamma_fm, beta_fm,
           W1, b1, gamma1, beta1, W2, b2):
    emb_flat = _tp_call(emb_table.T).reshape(PROWS, ED)
    lsum, pidx = _sc1_call(x.astype(jnp.int32).T, lin_table.reshape(-1))
    cross = _sc2_call(emb_flat, pidx)

    out = pl.pallas_call(
        _tc_body,
        out_shape=jax.ShapeDtypeStruct((B,), jnp.float32),
    )(cross, lsum, lin_bias, gamma_fm, beta_fm,
      W1, b1, gamma1, beta1, W2.reshape(1, HID), b2)
    return out
